# baseline (speedup 1.0000x reference)
import jax
import jax.numpy as jnp
from jax import lax
from jax.experimental import pallas as pl
from jax.experimental.pallas import tpu as pltpu
from jax.experimental.pallas import tpu_sc as plsc

B, H, D = 8, 16, 128
CAP = 2048
CHUNK = 16
CAPT = CAP // CHUNK

NC, NS = 2, 16
NW = NC * NS
TOTAL_ROWS = B * H * CAP
RPW = TOTAL_ROWS // NW
CH = 256
NCHUNK = RPW // CH
NBUF = 3

CHV = 2 * CAP
NCHV = TOTAL_ROWS // CHV
NBV = 8


def _sc_body(k_hbm, kval_hbm, pos_hbm, ko_hbm,
             buf, ld_sem, st_sem, kval_v, pos_v, idx_v, rs_sem):
    c = lax.axis_index("c")
    s = lax.axis_index("s")
    wid = c * NS + s
    base = wid * RPW

    for b in range(min(NBUF, NCHUNK)):
        pltpu.make_async_copy(k_hbm.at[pl.ds(base + b * CH, CH)],
                              buf.at[b], ld_sem.at[b]).start()
    for g in range(NCHUNK):
        b = g % NBUF
        pltpu.make_async_copy(k_hbm.at[pl.ds(base + g * CH, CH)],
                              buf.at[b], ld_sem.at[b]).wait()
        pltpu.make_async_copy(buf.at[b],
                              ko_hbm.at[pl.ds(base + g * CH, CH)],
                              st_sem.at[b]).start()
        nxt = g + 1
        if NBUF <= nxt < NCHUNK:
            bn = nxt % NBUF
            pltpu.make_async_copy(buf.at[bn],
                                  ko_hbm.at[pl.ds(base + (nxt - NBUF) * CH, CH)],
                                  st_sem.at[bn]).wait()
            pltpu.make_async_copy(k_hbm.at[pl.ds(base + nxt * CH, CH)],
                                  buf.at[bn], ld_sem.at[bn]).start()
    for g in range(max(0, NCHUNK - NBUF), NCHUNK):
        b = g % NBUF
        pltpu.make_async_copy(buf.at[b],
                              ko_hbm.at[pl.ds(base + g * CH, CH)],
                              st_sem.at[b]).wait()

    pltpu.make_async_copy(pos_hbm, pos_v, rs_sem).start()
    pltpu.make_async_copy(pos_hbm, pos_v, rs_sem).wait()
    for r in range(4):
        cp = pltpu.make_async_copy(kval_hbm.at[pl.ds(wid * 4, 4)],
                                   kval_v.at[pl.ds(4 * r, 4)], rs_sem)
        cp.start()
        cp.wait()
    lane = lax.iota(jnp.int32, 16)
    bh = wid * 4 + lax.rem(lane, 4)
    idx_v[...] = bh * CAP + pos_v[...]
    sc = pltpu.make_async_copy(kval_v, ko_hbm.at[idx_v], rs_sem)
    sc.start()
    sc.wait()


_sc_copy = pl.kernel(
    _sc_body,
    out_type=jax.ShapeDtypeStruct((TOTAL_ROWS, D), jnp.float32),
    mesh=plsc.VectorSubcoreMesh(core_axis_name="c", subcore_axis_name="s",
                                num_cores=NC, num_subcores=NS),
    scratch_types=[
        pltpu.VMEM((NBUF, CH, D), jnp.float32),
        pltpu.SemaphoreType.DMA((NBUF,)),
        pltpu.SemaphoreType.DMA((NBUF,)),
        pltpu.VMEM((16, D), jnp.float32),
        pltpu.VMEM((16,), jnp.int32),
        pltpu.VMEM((16,), jnp.int32),
        pltpu.SemaphoreType.DMA,
    ],
)


def _tc_v_body(pos_ref,
               vval_ref,
               vc_hbm, vo_hbm,
               buf, ld_sem, st_sem):
    pos = pos_ref[0]
    for b in range(NBV):
        pltpu.make_async_copy(vc_hbm.at[pl.ds(b * CHV, CHV)],
                              buf.at[b], ld_sem.at[b]).start()
    for g in range(NCHV):
        b = g % NBV
        pltpu.make_async_copy(vc_hbm.at[pl.ds(g * CHV, CHV)],
                              buf.at[b], ld_sem.at[b]).wait()
        buf[b, pl.ds(pos, 1), :] = vval_ref[pl.ds(2 * g, 1), 0, :]
        buf[b, pl.ds(CAP + pos, 1), :] = vval_ref[pl.ds(2 * g + 1, 1), 0, :]
        pltpu.make_async_copy(buf.at[b],
                              vo_hbm.at[pl.ds(g * CHV, CHV)],
                              st_sem.at[b]).start()
        nxt = g + 1
        if NBV <= nxt < NCHV:
            bn = nxt % NBV
            pltpu.make_async_copy(buf.at[bn],
                                  vo_hbm.at[pl.ds((nxt - NBV) * CHV, CHV)],
                                  st_sem.at[bn]).wait()
            pltpu.make_async_copy(vc_hbm.at[pl.ds(nxt * CHV, CHV)],
                                  buf.at[bn], ld_sem.at[bn]).start()
    for g in range(NCHV - NBV, NCHV):
        b = g % NBV
        pltpu.make_async_copy(buf.at[b],
                              vo_hbm.at[pl.ds(g * CHV, CHV)],
                              st_sem.at[b]).wait()


def _tc_kt_body(pos_ref,
                kval_ref,
                kt_ref,
                kt_out_ref):
    pos = pos_ref[0]
    kt = kt_ref[0]
    t = jnp.swapaxes(kt, -1, -2)
    kv = kval_ref[0, :, 0, :]
    kv2 = jnp.concatenate([kv, kv], axis=-1)[:, None, :]
    col = lax.broadcasted_iota(jnp.int32, (NKT, CAPT, 2 * D), 2)
    row = lax.broadcasted_iota(jnp.int32, (NKT, CAPT, 2 * D), 1)
    merged = jnp.where(col < D, jnp.minimum(t, kv2), jnp.maximum(t, kv2))
    chunk_idx = pos // CHUNK
    kt_out_ref[0] = jnp.where(row == chunk_idx, merged, t)


NKT = 16


def kernel(input_pos, q, k_val, v_val, k_cache, kt_cache, v_cache):
    del q
    pos32 = input_pos.astype(jnp.int32)
    pos16 = jnp.broadcast_to(pos32, (16,))

    k_out = _sc_copy(k_cache.reshape(TOTAL_ROWS, D),
                     k_val.reshape(B * H, D), pos16)

    v_out = pl.pallas_call(
        _tc_v_body,
        in_specs=[
            pl.BlockSpec(memory_space=pltpu.SMEM),
            pl.BlockSpec(memory_space=pltpu.VMEM),
            pl.BlockSpec(memory_space=pltpu.HBM),
        ],
        out_specs=pl.BlockSpec(memory_space=pltpu.HBM),
        out_shape=jax.ShapeDtypeStruct((TOTAL_ROWS, D), jnp.float32),
        scratch_shapes=[
            pltpu.VMEM((NBV, CHV, D), jnp.float32),
            pltpu.SemaphoreType.DMA((NBV,)),
            pltpu.SemaphoreType.DMA((NBV,)),
        ],
    )(pos32, v_val.reshape(B * H, 1, D), v_cache.reshape(TOTAL_ROWS, D))

    kt_out = pl.pallas_call(
        _tc_kt_body,
        grid=(B,),
        in_specs=[
            pl.BlockSpec(memory_space=pltpu.SMEM),
            pl.BlockSpec((1, NKT, 1, D), lambda i: (i, 0, 0, 0)),
            pl.BlockSpec((1, NKT, 2 * D, CAPT),
                         lambda i: (i, 0, 0, 0)),
        ],
        out_specs=pl.BlockSpec((1, NKT, CAPT, 2 * D),
                               lambda i: (i, 0, 0, 0)),
        out_shape=jax.ShapeDtypeStruct((B, H, CAPT, 2 * D), jnp.float32),
        compiler_params=pltpu.CompilerParams(
            dimension_semantics=("parallel",),
        ),
    )(pos32, k_val, kt_cache)

    return (kt_out, k_out.reshape(B, H, CAP, D),
            v_out.reshape(B, H, CAP, D))


# speedup vs baseline: 1.1781x; 1.1781x over previous
import jax
import jax.numpy as jnp
from jax import lax
from jax.experimental import pallas as pl
from jax.experimental.pallas import tpu as pltpu
from jax.experimental.pallas import tpu_sc as plsc

B, H, D = 8, 16, 128
CAP = 2048
CHUNK = 16
CAPT = CAP // CHUNK

NC, NS = 2, 16
NW = NC * NS
TOTAL_ROWS = B * H * CAP
RPW = TOTAL_ROWS // NW
CH = 256
NCHUNK = RPW // CH
NBUF = 3

CHV = 2 * CAP
NCHV = TOTAL_ROWS // CHV
NBV = 8


def _sc_body(k_hbm, kval_hbm, pos_hbm, ko_hbm,
             buf, ld_sem, st_sem, kval_v, pos_v, idx_v, rs_sem):
    c = lax.axis_index("c")
    s = lax.axis_index("s")
    wid = c * NS + s
    base = wid * RPW

    for b in range(min(NBUF, NCHUNK)):
        pltpu.make_async_copy(k_hbm.at[pl.ds(base + b * CH, CH)],
                              buf.at[b], ld_sem.at[b]).start()
    for g in range(NCHUNK):
        b = g % NBUF
        pltpu.make_async_copy(k_hbm.at[pl.ds(base + g * CH, CH)],
                              buf.at[b], ld_sem.at[b]).wait()
        pltpu.make_async_copy(buf.at[b],
                              ko_hbm.at[pl.ds(base + g * CH, CH)],
                              st_sem.at[b]).start()
        nxt = g + 1
        if NBUF <= nxt < NCHUNK:
            bn = nxt % NBUF
            pltpu.make_async_copy(buf.at[bn],
                                  ko_hbm.at[pl.ds(base + (nxt - NBUF) * CH, CH)],
                                  st_sem.at[bn]).wait()
            pltpu.make_async_copy(k_hbm.at[pl.ds(base + nxt * CH, CH)],
                                  buf.at[bn], ld_sem.at[bn]).start()
    for g in range(max(0, NCHUNK - NBUF), NCHUNK):
        b = g % NBUF
        pltpu.make_async_copy(buf.at[b],
                              ko_hbm.at[pl.ds(base + g * CH, CH)],
                              st_sem.at[b]).wait()

    pltpu.make_async_copy(pos_hbm, pos_v, rs_sem).start()
    pltpu.make_async_copy(pos_hbm, pos_v, rs_sem).wait()
    for r in range(4):
        cp = pltpu.make_async_copy(kval_hbm.at[pl.ds(wid * 4, 4)],
                                   kval_v.at[pl.ds(4 * r, 4)], rs_sem)
        cp.start()
        cp.wait()
    lane = lax.iota(jnp.int32, 16)
    bh = wid * 4 + lax.rem(lane, 4)
    idx_v[...] = bh * CAP + pos_v[...]
    sc = pltpu.make_async_copy(kval_v, ko_hbm.at[idx_v], rs_sem)
    sc.start()
    sc.wait()


_sc_copy = pl.kernel(
    _sc_body,
    out_type=jax.ShapeDtypeStruct((TOTAL_ROWS, D), jnp.float32),
    mesh=plsc.VectorSubcoreMesh(core_axis_name="c", subcore_axis_name="s",
                                num_cores=NC, num_subcores=NS),
    scratch_types=[
        pltpu.VMEM((NBUF, CH, D), jnp.float32),
        pltpu.SemaphoreType.DMA((NBUF,)),
        pltpu.SemaphoreType.DMA((NBUF,)),
        pltpu.VMEM((16, D), jnp.float32),
        pltpu.VMEM((16,), jnp.int32),
        pltpu.VMEM((16,), jnp.int32),
        pltpu.SemaphoreType.DMA,
    ],
)


NVH = 4


def _tc_v_body(pos_ref,
               vval_ref,
               vc_ref,
               vo_ref):
    i = pl.program_id(0)
    pos = pos_ref[0]
    vv = vval_ref[pl.ds(i * NVH, NVH), 0, :][:, None, :]
    r = lax.broadcasted_iota(jnp.int32, (NVH, CAP, D), 1)
    vo_ref[...] = jnp.where(r == pos, vv, vc_ref[...])


def _tc_kt_body(pos_ref,
                kval_ref,
                kt_ref,
                kt_out_ref):
    pos = pos_ref[0]
    kt = kt_ref[0]
    t = jnp.swapaxes(kt, -1, -2)
    kv = kval_ref[0, :, 0, :]
    kv2 = jnp.concatenate([kv, kv], axis=-1)[:, None, :]
    col = lax.broadcasted_iota(jnp.int32, (NKT, CAPT, 2 * D), 2)
    row = lax.broadcasted_iota(jnp.int32, (NKT, CAPT, 2 * D), 1)
    merged = jnp.where(col < D, jnp.minimum(t, kv2), jnp.maximum(t, kv2))
    chunk_idx = pos // CHUNK
    kt_out_ref[0] = jnp.where(row == chunk_idx, merged, t)


NKT = 16


def kernel(input_pos, q, k_val, v_val, k_cache, kt_cache, v_cache):
    del q
    pos32 = input_pos.astype(jnp.int32)
    pos16 = jnp.broadcast_to(pos32, (16,))

    k_out = _sc_copy(k_cache.reshape(TOTAL_ROWS, D),
                     k_val.reshape(B * H, D), pos16)

    v_out = pl.pallas_call(
        _tc_v_body,
        grid=(B * H // NVH,),
        in_specs=[
            pl.BlockSpec(memory_space=pltpu.SMEM),
            pl.BlockSpec(memory_space=pltpu.VMEM),
            pl.BlockSpec((NVH, CAP, D), lambda i: (i, 0, 0)),
        ],
        out_specs=pl.BlockSpec((NVH, CAP, D), lambda i: (i, 0, 0)),
        out_shape=jax.ShapeDtypeStruct((B * H, CAP, D), jnp.float32),
        compiler_params=pltpu.CompilerParams(
            dimension_semantics=("parallel",),
        ),
    )(pos32, v_val.reshape(B * H, 1, D), v_cache.reshape(B * H, CAP, D))

    kt_out = pl.pallas_call(
        _tc_kt_body,
        grid=(B,),
        in_specs=[
            pl.BlockSpec(memory_space=pltpu.SMEM),
            pl.BlockSpec((1, NKT, 1, D), lambda i: (i, 0, 0, 0)),
            pl.BlockSpec((1, NKT, 2 * D, CAPT),
                         lambda i: (i, 0, 0, 0)),
        ],
        out_specs=pl.BlockSpec((1, NKT, CAPT, 2 * D),
                               lambda i: (i, 0, 0, 0)),
        out_shape=jax.ShapeDtypeStruct((B, H, CAPT, 2 * D), jnp.float32),
        compiler_params=pltpu.CompilerParams(
            dimension_semantics=("parallel",),
        ),
    )(pos32, k_val, kt_cache)

    return (kt_out, k_out.reshape(B, H, CAP, D),
            v_out.reshape(B, H, CAP, D))


# speedup vs baseline: 1.1837x; 1.0048x over previous
import jax
import jax.numpy as jnp
from jax import lax
from jax.experimental import pallas as pl
from jax.experimental.pallas import tpu as pltpu
from jax.experimental.pallas import tpu_sc as plsc

B, H, D = 8, 16, 128
CAP = 2048
CHUNK = 16
CAPT = CAP // CHUNK

NC, NS = 2, 16
NW = NC * NS
TOTAL_ROWS = B * H * CAP
RPW = TOTAL_ROWS // NW
CH = 256
NCHUNK = RPW // CH
NBUF = 3

CHV = 2 * CAP
NCHV = TOTAL_ROWS // CHV
NBV = 8


def _sc_body(k_hbm, kval_hbm, pos_hbm, ko_hbm,
             buf, ld_sem, st_sem, kval_v, pos_v, idx_v, rs_sem):
    c = lax.axis_index("c")
    s = lax.axis_index("s")
    wid = c * NS + s
    base = wid * RPW

    for b in range(min(NBUF, NCHUNK)):
        pltpu.make_async_copy(k_hbm.at[pl.ds(base + b * CH, CH)],
                              buf.at[b], ld_sem.at[b]).start()
    for g in range(NCHUNK):
        b = g % NBUF
        pltpu.make_async_copy(k_hbm.at[pl.ds(base + g * CH, CH)],
                              buf.at[b], ld_sem.at[b]).wait()
        pltpu.make_async_copy(buf.at[b],
                              ko_hbm.at[pl.ds(base + g * CH, CH)],
                              st_sem.at[b]).start()
        nxt = g + 1
        if NBUF <= nxt < NCHUNK:
            bn = nxt % NBUF
            pltpu.make_async_copy(buf.at[bn],
                                  ko_hbm.at[pl.ds(base + (nxt - NBUF) * CH, CH)],
                                  st_sem.at[bn]).wait()
            pltpu.make_async_copy(k_hbm.at[pl.ds(base + nxt * CH, CH)],
                                  buf.at[bn], ld_sem.at[bn]).start()
    for g in range(max(0, NCHUNK - NBUF), NCHUNK):
        b = g % NBUF
        pltpu.make_async_copy(buf.at[b],
                              ko_hbm.at[pl.ds(base + g * CH, CH)],
                              st_sem.at[b]).wait()

    pltpu.make_async_copy(pos_hbm, pos_v, rs_sem).start()
    pltpu.make_async_copy(pos_hbm, pos_v, rs_sem).wait()
    for r in range(4):
        cp = pltpu.make_async_copy(kval_hbm.at[pl.ds(wid * 4, 4)],
                                   kval_v.at[pl.ds(4 * r, 4)], rs_sem)
        cp.start()
        cp.wait()
    lane = lax.iota(jnp.int32, 16)
    bh = wid * 4 + lax.rem(lane, 4)
    idx_v[...] = bh * CAP + pos_v[...]
    sc = pltpu.make_async_copy(kval_v, ko_hbm.at[idx_v], rs_sem)
    sc.start()
    sc.wait()


_sc_copy = pl.kernel(
    _sc_body,
    out_type=jax.ShapeDtypeStruct((TOTAL_ROWS, D), jnp.float32),
    mesh=plsc.VectorSubcoreMesh(core_axis_name="c", subcore_axis_name="s",
                                num_cores=NC, num_subcores=NS),
    scratch_types=[
        pltpu.VMEM((NBUF, CH, D), jnp.float32),
        pltpu.SemaphoreType.DMA((NBUF,)),
        pltpu.SemaphoreType.DMA((NBUF,)),
        pltpu.VMEM((16, D), jnp.float32),
        pltpu.VMEM((16,), jnp.int32),
        pltpu.VMEM((16,), jnp.int32),
        pltpu.SemaphoreType.DMA,
    ],
)


NVH = 8


def _tc_v_body(pos_ref,
               vval_ref,
               vc_ref,
               vo_ref):
    i = pl.program_id(0)
    pos = pos_ref[0]
    vv = vval_ref[pl.ds(i * NVH, NVH), 0, :][:, None, :]
    r = lax.broadcasted_iota(jnp.int32, (NVH, CAP, D), 1)
    vo_ref[...] = jnp.where(r == pos, vv, vc_ref[...])


def _tc_kt_body(pos_ref,
                kval_ref,
                kt_ref,
                kt_out_ref):
    pos = pos_ref[0]
    kt = kt_ref[0]
    t = jnp.swapaxes(kt, -1, -2)
    kv = kval_ref[0, :, 0, :]
    kv2 = jnp.concatenate([kv, kv], axis=-1)[:, None, :]
    col = lax.broadcasted_iota(jnp.int32, (NKT, CAPT, 2 * D), 2)
    row = lax.broadcasted_iota(jnp.int32, (NKT, CAPT, 2 * D), 1)
    merged = jnp.where(col < D, jnp.minimum(t, kv2), jnp.maximum(t, kv2))
    chunk_idx = pos // CHUNK
    kt_out_ref[0] = jnp.where(row == chunk_idx, merged, t)


NKT = 16


def kernel(input_pos, q, k_val, v_val, k_cache, kt_cache, v_cache):
    del q
    pos32 = input_pos.astype(jnp.int32)
    pos16 = jnp.broadcast_to(pos32, (16,))

    k_out = _sc_copy(k_cache.reshape(TOTAL_ROWS, D),
                     k_val.reshape(B * H, D), pos16)

    v_out = pl.pallas_call(
        _tc_v_body,
        grid=(B * H // NVH,),
        in_specs=[
            pl.BlockSpec(memory_space=pltpu.SMEM),
            pl.BlockSpec(memory_space=pltpu.VMEM),
            pl.BlockSpec((NVH, CAP, D), lambda i: (i, 0, 0)),
        ],
        out_specs=pl.BlockSpec((NVH, CAP, D), lambda i: (i, 0, 0)),
        out_shape=jax.ShapeDtypeStruct((B * H, CAP, D), jnp.float32),
        compiler_params=pltpu.CompilerParams(
            dimension_semantics=("parallel",),
        ),
    )(pos32, v_val.reshape(B * H, 1, D), v_cache.reshape(B * H, CAP, D))

    kt_out = pl.pallas_call(
        _tc_kt_body,
        grid=(B,),
        in_specs=[
            pl.BlockSpec(memory_space=pltpu.SMEM),
            pl.BlockSpec((1, NKT, 1, D), lambda i: (i, 0, 0, 0)),
            pl.BlockSpec((1, NKT, 2 * D, CAPT),
                         lambda i: (i, 0, 0, 0)),
        ],
        out_specs=pl.BlockSpec((1, NKT, CAPT, 2 * D),
                               lambda i: (i, 0, 0, 0)),
        out_shape=jax.ShapeDtypeStruct((B, H, CAPT, 2 * D), jnp.float32),
        compiler_params=pltpu.CompilerParams(
            dimension_semantics=("parallel",),
        ),
    )(pos32, k_val, kt_cache)

    return (kt_out, k_out.reshape(B, H, CAP, D),
            v_out.reshape(B, H, CAP, D))
